# baseline (speedup 1.0000x reference)
import functools
import math

import jax
import jax.numpy as jnp
from jax import lax
from jax.experimental import pallas as pl
from jax.experimental.pallas import tpu as pltpu
from jax.experimental.pallas import tpu_sc as plsc

VOCAB = 100000
EMB = 64
N_BASE = 16384
N_POS = 8192
N_OUT = N_BASE + N_POS

NC, NS, L = 2, 16, 16
NW = NC * NS
CPW = EMB // NW
CHUNK = 1024
NCHUNK = N_OUT // CHUNK
NBUF = 3
LT = 128
VFULL = VOCAB // LT
VPAD = (VFULL + 1) * LT
COLS = VPAD + N_POS


def _posenc_body(pt_ref, out_ref):
    pt = pt_ref[...].astype(jnp.float32)[None, :]
    row = lax.broadcasted_iota(jnp.int32, (EMB, 1), 0)
    k = (row % (EMB // 2)).astype(jnp.float32)
    period = jnp.exp(k * (-2.0 * math.log(10000.0) / EMB))
    shift = jnp.where(row < EMB // 2, 0.0, 0.5 * math.pi)
    out_ref[...] = jnp.sin(pt * period + shift)


_posenc = pl.pallas_call(
    _posenc_body,
    out_shape=jax.ShapeDtypeStruct((EMB, N_POS), jnp.float32),
)


GU = 8


def _sc_body(bmt_hbm, idx_hbm, pe_hbm, tt_hbm, tail_hbm, out_hbm,
             bmt_v, col_v, idx_v0, idx_v1, idx_v2, out_v0, out_v1, out_v2,
             lsem, isem, wsem):
    wid = lax.axis_index("s") * NC + lax.axis_index("c")
    idx_bufs = [idx_v0, idx_v1, idx_v2]
    out_bufs = [out_v0, out_v1, out_v2]
    pltpu.sync_copy(bmt_hbm, bmt_v)

    def do_column(cix):
        def fire_tt(g, carry):
            o = g * (4 * LT)
            for u in range(4):
                pltpu.async_copy(tt_hbm.at[cix, pl.ds(o + u * LT, LT)],
                                 col_v.at[pl.ds(o + u * LT, LT)], lsem)
            return carry
        lax.fori_loop(0, VFULL // 4, fire_tt, 0)
        pltpu.async_copy(tt_hbm.at[cix, pl.ds((VFULL - 1) * LT, LT)],
                         col_v.at[pl.ds((VFULL - 1) * LT, LT)], lsem)
        pltpu.async_copy(tail_hbm.at[cix],
                         col_v.at[pl.ds(VFULL * LT, LT)], lsem)

        def fire_pe(g, carry):
            o = g * (4 * LT)
            for u in range(4):
                pltpu.async_copy(
                    pe_hbm.at[cix, pl.ds(o + u * LT, LT)],
                    col_v.at[pl.ds(VPAD + o + u * LT, LT)], lsem)
            return carry
        lax.fori_loop(0, N_POS // (4 * LT), fire_pe, 0)
        for p in range(2):
            pltpu.async_copy(idx_hbm.at[pl.ds(p * CHUNK, CHUNK)],
                             idx_bufs[p], isem)
        for _ in range(4):
            pltpu.make_async_copy(out_hbm.at[cix],
                                  col_v.at[pl.ds(0, N_OUT)], lsem).wait()
        pltpu.make_async_copy(
            out_hbm.at[cix, pl.ds(0, COLS - 4 * N_OUT)],
            col_v.at[pl.ds(0, COLS - 4 * N_OUT)], lsem).wait()

        for ch in range(NCHUNK):
            idx_v = idx_bufs[ch % NBUF]
            out_v = out_bufs[ch % NBUF]
            pltpu.make_async_copy(idx_hbm.at[pl.ds(0, CHUNK)], idx_v,
                                  isem).wait()
            if ch + 2 < NCHUNK:
                pltpu.async_copy(
                    idx_hbm.at[pl.ds((ch + 2) * CHUNK, CHUNK)],
                    idx_bufs[(ch + 2) % NBUF], isem)
            if ch >= NBUF:
                pltpu.make_async_copy(out_v, out_hbm.at[cix, pl.ds(0, CHUNK)],
                                      wsem).wait()

            def grp(g, carry):
                base = g * (GU * L)
                for u in range(GU):
                    sl = pl.ds(base + u * L, L)
                    iv = idx_v[sl]
                    isb = iv < N_BASE
                    tok = plsc.load_gather(bmt_v, [lax.rem(iv, N_BASE)])
                    comb = jnp.where(isb, tok, iv + (VPAD - N_BASE))
                    out_v[sl] = plsc.load_gather(col_v, [comb])
                return carry
            lax.fori_loop(0, CHUNK // (GU * L), grp, 0)

            def fire_out(g, carry):
                o = g * (4 * LT)
                for u in range(4):
                    pltpu.async_copy(
                        out_v.at[pl.ds(o + u * LT, LT)],
                        out_hbm.at[cix, pl.ds(ch * CHUNK + o + u * LT, LT)],
                        wsem)
                return carry
            lax.fori_loop(0, CHUNK // (4 * LT), fire_out, 0)

        for b in range(NBUF):
            pltpu.make_async_copy(out_bufs[b],
                                  out_hbm.at[cix, pl.ds(0, CHUNK)],
                                  wsem).wait()

    for q in range(CPW):
        do_column(wid * CPW + q)


_sc_call = functools.partial(
    pl.kernel,
    out_type=jax.ShapeDtypeStruct((EMB, N_OUT), jnp.float32),
    mesh=plsc.VectorSubcoreMesh(core_axis_name="c", subcore_axis_name="s"),
    compiler_params=pltpu.CompilerParams(needs_layout_passes=False,
                                         use_tc_tiling_on_sc=True),
    scratch_types=[
        pltpu.VMEM((N_BASE,), jnp.int32),
        pltpu.VMEM((COLS,), jnp.float32),
        pltpu.VMEM((CHUNK,), jnp.int32),
        pltpu.VMEM((CHUNK,), jnp.int32),
        pltpu.VMEM((CHUNK,), jnp.int32),
        pltpu.VMEM((CHUNK,), jnp.float32),
        pltpu.VMEM((CHUNK,), jnp.float32),
        pltpu.VMEM((CHUNK,), jnp.float32),
        pltpu.SemaphoreType.DMA,
        pltpu.SemaphoreType.DMA,
        pltpu.SemaphoreType.DMA,
    ],
)(_sc_body)


def kernel(base_model_tokens, positional_tokens, base_idxs_of_tokens, table):
    pe_t = _posenc(positional_tokens.astype(jnp.int32))
    tail = jnp.pad(table[VFULL * LT:].T, ((0, 0), (0, VPAD - VOCAB)))
    out_t = _sc_call(base_model_tokens.astype(jnp.int32),
                     base_idxs_of_tokens.astype(jnp.int32), pe_t, table.T,
                     tail)
    return out_t.T


# speedup vs baseline: 1.4917x; 1.4917x over previous
import functools
import math

import jax
import jax.numpy as jnp
from jax import lax
from jax.experimental import pallas as pl
from jax.experimental.pallas import tpu as pltpu
from jax.experimental.pallas import tpu_sc as plsc

VOCAB = 100000
EMB = 64
N_BASE = 16384
N_POS = 8192
N_OUT = N_BASE + N_POS

NC, NS, L = 2, 16, 16
NW = NC * NS
CPW = EMB // NW
CHUNK = 4096
NCHUNK = N_OUT // CHUNK
NBUF = 2
RSL = N_OUT // NS
LT = 128
VFULL = VOCAB // LT
VPAD = (VFULL + 1) * LT
COLS = VPAD + N_POS


def _posenc_body(pt_ref, out_ref):
    pt = pt_ref[...].astype(jnp.float32)[None, :]
    row = lax.broadcasted_iota(jnp.int32, (EMB, 1), 0)
    k = (row % (EMB // 2)).astype(jnp.float32)
    period = jnp.exp(k * (-2.0 * math.log(10000.0) / EMB))
    shift = jnp.where(row < EMB // 2, 0.0, 0.5 * math.pi)
    out_ref[...] = jnp.sin(pt * period + shift)


_posenc = pl.pallas_call(
    _posenc_body,
    out_shape=jax.ShapeDtypeStruct((EMB, N_POS), jnp.float32),
)


GU = 8


def _sc_body(bmt_hbm, idx_hbm, pe_hbm, tt_hbm, tail_hbm, out_hbm,
             col_v, idx_v0, idx_v1, out_v0, out_v1, comb_spm,
             lsem, isem, wsem):
    sid = lax.axis_index("s")
    wid = sid * NC + lax.axis_index("c")
    idx_bufs = [idx_v0, idx_v1]
    out_bufs = [out_v0, out_v1]

    bmt_f = col_v.at[pl.ds(0, N_BASE)]
    pltpu.sync_copy(bmt_hbm, bmt_f)
    pltpu.sync_copy(idx_hbm.at[pl.ds(sid * RSL, RSL)],
                    idx_v0.at[pl.ds(0, RSL)])

    def resolve(g, carry):
        base = g * (GU * L)
        for u in range(GU):
            sl = pl.ds(base + u * L, L)
            iv = idx_v0[sl]
            isb = iv < N_BASE
            tokf = plsc.load_gather(bmt_f, [lax.rem(iv, N_BASE)])
            tok = plsc.bitcast(tokf, jnp.int32)
            idx_v1[sl] = jnp.where(isb, tok, iv + (VPAD - N_BASE))
        return carry
    lax.fori_loop(0, RSL // (GU * L), resolve, 0)
    pltpu.sync_copy(idx_v1.at[pl.ds(0, RSL)],
                    comb_spm.at[pl.ds(sid * RSL, RSL)])
    plsc.subcore_barrier()

    def do_column(cix):
        def fire_tt(g, carry):
            o = g * (4 * LT)
            for u in range(4):
                pltpu.async_copy(tt_hbm.at[cix, pl.ds(o + u * LT, LT)],
                                 col_v.at[pl.ds(o + u * LT, LT)], lsem)
            return carry
        lax.fori_loop(0, VFULL // 4, fire_tt, 0)
        pltpu.async_copy(tt_hbm.at[cix, pl.ds((VFULL - 1) * LT, LT)],
                         col_v.at[pl.ds((VFULL - 1) * LT, LT)], lsem)
        pltpu.async_copy(tail_hbm.at[cix],
                         col_v.at[pl.ds(VFULL * LT, LT)], lsem)

        def fire_pe(g, carry):
            o = g * (4 * LT)
            for u in range(4):
                pltpu.async_copy(
                    pe_hbm.at[cix, pl.ds(o + u * LT, LT)],
                    col_v.at[pl.ds(VPAD + o + u * LT, LT)], lsem)
            return carry
        lax.fori_loop(0, N_POS // (4 * LT), fire_pe, 0)
        pltpu.async_copy(comb_spm.at[pl.ds(0, CHUNK)], idx_bufs[0], isem)
        for _ in range(4):
            pltpu.make_async_copy(out_hbm.at[cix],
                                  col_v.at[pl.ds(0, N_OUT)], lsem).wait()
        pltpu.make_async_copy(
            out_hbm.at[cix, pl.ds(0, COLS - 4 * N_OUT)],
            col_v.at[pl.ds(0, COLS - 4 * N_OUT)], lsem).wait()

        for ch in range(NCHUNK):
            idx_v = idx_bufs[ch % NBUF]
            out_v = out_bufs[ch % NBUF]
            pltpu.make_async_copy(comb_spm.at[pl.ds(0, CHUNK)], idx_v,
                                  isem).wait()
            if ch + 1 < NCHUNK:
                pltpu.async_copy(
                    comb_spm.at[pl.ds((ch + 1) * CHUNK, CHUNK)],
                    idx_bufs[(ch + 1) % NBUF], isem)
            if ch >= NBUF:
                pltpu.make_async_copy(out_v, out_hbm.at[cix, pl.ds(0, CHUNK)],
                                      wsem).wait()

            def grp(g, carry):
                base = g * (GU * L)
                for u in range(GU):
                    sl = pl.ds(base + u * L, L)
                    out_v[sl] = plsc.load_gather(col_v, [idx_v[sl]])
                return carry
            lax.fori_loop(0, CHUNK // (GU * L), grp, 0)

            def fire_out(g, carry):
                o = g * (4 * LT)
                for u in range(4):
                    pltpu.async_copy(
                        out_v.at[pl.ds(o + u * LT, LT)],
                        out_hbm.at[cix, pl.ds(ch * CHUNK + o + u * LT, LT)],
                        wsem)
                return carry
            lax.fori_loop(0, CHUNK // (4 * LT), fire_out, 0)

        for b in range(NBUF):
            pltpu.make_async_copy(out_bufs[b],
                                  out_hbm.at[cix, pl.ds(0, CHUNK)],
                                  wsem).wait()

    for q in range(CPW):
        do_column(wid * CPW + q)


_sc_call = functools.partial(
    pl.kernel,
    out_type=jax.ShapeDtypeStruct((EMB, N_OUT), jnp.float32),
    mesh=plsc.VectorSubcoreMesh(core_axis_name="c", subcore_axis_name="s"),
    compiler_params=pltpu.CompilerParams(needs_layout_passes=False,
                                         use_tc_tiling_on_sc=True),
    scratch_types=[
        pltpu.VMEM((COLS,), jnp.float32),
        pltpu.VMEM((CHUNK,), jnp.int32),
        pltpu.VMEM((CHUNK,), jnp.int32),
        pltpu.VMEM((CHUNK,), jnp.float32),
        pltpu.VMEM((CHUNK,), jnp.float32),
        pltpu.VMEM_SHARED((N_OUT,), jnp.int32),
        pltpu.SemaphoreType.DMA,
        pltpu.SemaphoreType.DMA,
        pltpu.SemaphoreType.DMA,
    ],
)(_sc_body)


def kernel(base_model_tokens, positional_tokens, base_idxs_of_tokens, table):
    pe_t = _posenc(positional_tokens.astype(jnp.int32))
    tail = jnp.pad(table[VFULL * LT:].T, ((0, 0), (0, VPAD - VOCAB)))
    bmt_f = lax.bitcast_convert_type(base_model_tokens.astype(jnp.int32),
                                     jnp.float32)
    out_t = _sc_call(bmt_f, base_idxs_of_tokens.astype(jnp.int32), pe_t,
                     table.T, tail)
    return out_t.T


# speedup vs baseline: 1.5001x; 1.0056x over previous
import functools
import math

import jax
import jax.numpy as jnp
from jax import lax
from jax.experimental import pallas as pl
from jax.experimental.pallas import tpu as pltpu
from jax.experimental.pallas import tpu_sc as plsc

VOCAB = 100000
EMB = 64
N_BASE = 16384
N_POS = 8192
N_OUT = N_BASE + N_POS

NC, NS, L = 2, 16, 16
NW = NC * NS
CPW = EMB // NW
CHUNK = 4096
NCHUNK = N_OUT // CHUNK
NBUF = 2
RSL = N_OUT // NS
LT = 128
VFULL = VOCAB // LT
VPAD = (VFULL + 1) * LT
COLS = VPAD + N_POS


PE_B = N_POS // 8


def _posenc_body(pt_ref, out_ref):
    pt = pt_ref[...].astype(jnp.float32)[None, :]
    row = lax.broadcasted_iota(jnp.int32, (EMB, 1), 0)
    k = (row % (EMB // 2)).astype(jnp.float32)
    period = jnp.exp(k * (-2.0 * math.log(10000.0) / EMB))
    shift = jnp.where(row < EMB // 2, 0.0, 0.5 * math.pi)
    out_ref[...] = jnp.sin(pt * period + shift)


_posenc = pl.pallas_call(
    _posenc_body,
    grid=(N_POS // PE_B,),
    in_specs=[pl.BlockSpec((PE_B,), lambda g: (g,))],
    out_specs=pl.BlockSpec((EMB, PE_B), lambda g: (0, g)),
    out_shape=jax.ShapeDtypeStruct((EMB, N_POS), jnp.float32),
)


GU = 8


def _sc_body(bmt_hbm, idx_hbm, pe_hbm, tt_hbm, tail_hbm, out_hbm,
             col_v, idx_v0, idx_v1, out_v0, out_v1, comb_spm,
             lsem, isem, wsem):
    sid = lax.axis_index("s")
    wid = sid * NC + lax.axis_index("c")
    idx_bufs = [idx_v0, idx_v1]
    out_bufs = [out_v0, out_v1]

    bmt_f = col_v.at[pl.ds(0, N_BASE)]
    pltpu.sync_copy(bmt_hbm, bmt_f)
    pltpu.sync_copy(idx_hbm.at[pl.ds(sid * RSL, RSL)],
                    idx_v0.at[pl.ds(0, RSL)])

    def resolve(g, carry):
        base = g * (GU * L)
        for u in range(GU):
            sl = pl.ds(base + u * L, L)
            iv = idx_v0[sl]
            isb = iv < N_BASE
            tokf = plsc.load_gather(bmt_f, [lax.rem(iv, N_BASE)])
            tok = plsc.bitcast(tokf, jnp.int32)
            idx_v1[sl] = jnp.where(isb, tok, iv + (VPAD - N_BASE))
        return carry
    lax.fori_loop(0, RSL // (GU * L), resolve, 0)
    pltpu.sync_copy(idx_v1.at[pl.ds(0, RSL)],
                    comb_spm.at[pl.ds(sid * RSL, RSL)])
    plsc.subcore_barrier()

    def do_column(cix):
        def fire_tt(g, carry):
            o = g * (4 * LT)
            for u in range(4):
                pltpu.async_copy(tt_hbm.at[cix, pl.ds(o + u * LT, LT)],
                                 col_v.at[pl.ds(o + u * LT, LT)], lsem)
            return carry
        lax.fori_loop(0, VFULL // 4, fire_tt, 0)
        pltpu.async_copy(tt_hbm.at[cix, pl.ds((VFULL - 1) * LT, LT)],
                         col_v.at[pl.ds((VFULL - 1) * LT, LT)], lsem)
        pltpu.async_copy(tail_hbm.at[cix],
                         col_v.at[pl.ds(VFULL * LT, LT)], lsem)

        def fire_pe(g, carry):
            o = g * (4 * LT)
            for u in range(4):
                pltpu.async_copy(
                    pe_hbm.at[cix, pl.ds(o + u * LT, LT)],
                    col_v.at[pl.ds(VPAD + o + u * LT, LT)], lsem)
            return carry
        lax.fori_loop(0, N_POS // (4 * LT), fire_pe, 0)
        pltpu.async_copy(comb_spm.at[pl.ds(0, CHUNK)], idx_bufs[0], isem)
        for _ in range(4):
            pltpu.make_async_copy(out_hbm.at[cix],
                                  col_v.at[pl.ds(0, N_OUT)], lsem).wait()
        pltpu.make_async_copy(
            out_hbm.at[cix, pl.ds(0, COLS - 4 * N_OUT)],
            col_v.at[pl.ds(0, COLS - 4 * N_OUT)], lsem).wait()

        for ch in range(NCHUNK):
            idx_v = idx_bufs[ch % NBUF]
            out_v = out_bufs[ch % NBUF]
            pltpu.make_async_copy(comb_spm.at[pl.ds(0, CHUNK)], idx_v,
                                  isem).wait()
            if ch + 1 < NCHUNK:
                pltpu.async_copy(
                    comb_spm.at[pl.ds((ch + 1) * CHUNK, CHUNK)],
                    idx_bufs[(ch + 1) % NBUF], isem)
            if ch >= NBUF:
                pltpu.make_async_copy(out_v, out_hbm.at[cix, pl.ds(0, CHUNK)],
                                      wsem).wait()

            def grp(g, carry):
                base = g * (GU * L)
                for u in range(GU):
                    sl = pl.ds(base + u * L, L)
                    out_v[sl] = plsc.load_gather(col_v, [idx_v[sl]])
                return carry
            lax.fori_loop(0, CHUNK // (GU * L), grp, 0)

            def fire_out(g, carry):
                o = g * (4 * LT)
                for u in range(4):
                    pltpu.async_copy(
                        out_v.at[pl.ds(o + u * LT, LT)],
                        out_hbm.at[cix, pl.ds(ch * CHUNK + o + u * LT, LT)],
                        wsem)
                return carry
            lax.fori_loop(0, CHUNK // (4 * LT), fire_out, 0)

        for b in range(NBUF):
            pltpu.make_async_copy(out_bufs[b],
                                  out_hbm.at[cix, pl.ds(0, CHUNK)],
                                  wsem).wait()

    for q in range(CPW):
        do_column(wid * CPW + q)


_sc_call = functools.partial(
    pl.kernel,
    out_type=jax.ShapeDtypeStruct((EMB, N_OUT), jnp.float32),
    mesh=plsc.VectorSubcoreMesh(core_axis_name="c", subcore_axis_name="s"),
    compiler_params=pltpu.CompilerParams(needs_layout_passes=False,
                                         use_tc_tiling_on_sc=True),
    scratch_types=[
        pltpu.VMEM((COLS,), jnp.float32),
        pltpu.VMEM((CHUNK,), jnp.int32),
        pltpu.VMEM((CHUNK,), jnp.int32),
        pltpu.VMEM((CHUNK,), jnp.float32),
        pltpu.VMEM((CHUNK,), jnp.float32),
        pltpu.VMEM_SHARED((N_OUT,), jnp.int32),
        pltpu.SemaphoreType.DMA,
        pltpu.SemaphoreType.DMA,
        pltpu.SemaphoreType.DMA,
    ],
)(_sc_body)


def kernel(base_model_tokens, positional_tokens, base_idxs_of_tokens, table):
    pe_t = _posenc(positional_tokens.astype(jnp.int32))
    tail = jnp.pad(table[VFULL * LT:].T, ((0, 0), (0, VPAD - VOCAB)))
    bmt_f = lax.bitcast_convert_type(base_model_tokens.astype(jnp.int32),
                                     jnp.float32)
    out_t = _sc_call(bmt_f, base_idxs_of_tokens.astype(jnp.int32), pe_t,
                     table.T, tail)
    return out_t.T
